# baseline (speedup 1.0000x reference)
import functools

import jax
import jax.numpy as jnp
from jax import lax
from jax.experimental import pallas as pl
from jax.experimental.pallas import tpu as pltpu, tpu_sc as plsc

B = 16384
D_IN = 256
NUM_VALUES = 100000
NUM_TOKENS = 16
TOKEN_DIM = 64
LANES = 16

NC = 2
NS = 16
NW = NC * NS
B_PER_W = B // NW
CHUNK = 32
N_CHUNKS = B_PER_W // CHUNK
CH2 = 128
N_CHUNKS2 = B_PER_W // CH2

_GATHER_DNUMS = lax.GatherDimensionNumbers(
    offset_dims=(), collapsed_slice_dims=(0,), start_index_map=(0,))


def _bcast_lane(vec, t):
    idx = jnp.full((LANES, 1), t, jnp.int32)
    return lax.gather(vec, idx, _GATHER_DNUMS, (1,),
                      mode=lax.GatherScatterMode.PROMISE_IN_BOUNDS)


def _alpha_body(x_ref, w_ref, b_ref, o_ref):
    att = jnp.dot(x_ref[...], w_ref[...], preferred_element_type=jnp.float32)
    o_ref[...] = jax.nn.sigmoid(att + b_ref[...])


def _compute_alpha_packed(x, W_att, b_att):
    blk = 2048
    alpha = pl.pallas_call(
        _alpha_body,
        grid=(B // blk,),
        in_specs=[
            pl.BlockSpec((blk, D_IN), lambda i: (i, 0)),
            pl.BlockSpec((D_IN, NUM_TOKENS), lambda i: (0, 0)),
            pl.BlockSpec((1, NUM_TOKENS), lambda i: (0, 0)),
        ],
        out_specs=pl.BlockSpec((blk, NUM_TOKENS), lambda i: (i, 0)),
        out_shape=jax.ShapeDtypeStruct((B, NUM_TOKENS), jnp.float32),
    )(x, W_att, b_att.reshape(1, NUM_TOKENS))
    return alpha.reshape(B // 8, 8 * NUM_TOKENS)


def _k1_body(tok_hbm, idx_hbm, alpha_hbm, out_hbm,
             idx_v, al_v, out_v, tok_v0, tok_v1, tok_v2,
             sem_t0, sem_t1, sem_t2):
    wid = lax.axis_index("s") * NC + lax.axis_index("c")
    base = wid * B_PER_W
    pltpu.sync_copy(idx_hbm.at[pl.ds(base, B_PER_W)], idx_v)
    pltpu.sync_copy(
        alpha_hbm.at[pl.ds(pl.multiple_of(base // 8, 8), B_PER_W // 8)], al_v)

    tok_bufs = (tok_v0, tok_v1, tok_v2)
    sems_t = (sem_t0, sem_t1, sem_t2)

    def start(g):
        bpar = g % 3
        pltpu.async_copy(tok_hbm.at[idx_v.at[pl.ds(g * CHUNK, CHUNK)]],
                         tok_bufs[bpar], sems_t[bpar])

    def wait(g):
        bpar = g % 3
        pltpu.make_async_copy(tok_hbm.at[idx_v.at[pl.ds(g * CHUNK, CHUNK)]],
                              tok_bufs[bpar], sems_t[bpar]).wait()

    start(0)
    start(1)
    start(2)

    for g in range(N_CHUNKS):
        wait(g)
        tok_v = tok_bufs[g % 3]
        goff = g * CHUNK

        def row_body(r, c2, goff=goff, tok_v=tok_v):
            row = goff + r
            a_vec = al_v[lax.shift_right_logical(row, 3),
                         pl.ds((row & 7) * LANES, LANES)]
            a = [_bcast_lane(a_vec, t) for t in range(NUM_TOKENS)]
            half = (r & 1) * TOKEN_DIM
            for c in range(TOKEN_DIM // LANES):
                acc = a[0] * tok_v[r, pl.ds(c * LANES, LANES)]
                for t in range(1, NUM_TOKENS):
                    acc = acc + a[t] * tok_v[r, pl.ds(t * TOKEN_DIM + c * LANES, LANES)]
                out_v[lax.shift_right_logical(r, 1),
                      pl.ds(half + c * LANES, LANES)] = acc
            return c2

        lax.fori_loop(0, CHUNK, row_body, 0)
        pltpu.sync_copy(
            out_v,
            out_hbm.at[pl.ds(pl.multiple_of((base + goff) // 2, 8),
                             CHUNK // 2)])
        if g + 3 < N_CHUNKS:
            start(g + 3)


_k1 = functools.partial(
    pl.kernel,
    out_type=jax.ShapeDtypeStruct((B // 2, 2 * TOKEN_DIM), jnp.float32),
    mesh=plsc.VectorSubcoreMesh(core_axis_name="c", subcore_axis_name="s"),
    scratch_types=[
        pltpu.VMEM((B_PER_W,), jnp.int32),
        pltpu.VMEM((B_PER_W // 8, 8 * NUM_TOKENS), jnp.float32),
        pltpu.VMEM((CHUNK // 2, 2 * TOKEN_DIM), jnp.float32),
        pltpu.VMEM((CHUNK, NUM_TOKENS * TOKEN_DIM), jnp.float32),
        pltpu.VMEM((CHUNK, NUM_TOKENS * TOKEN_DIM), jnp.float32),
        pltpu.VMEM((CHUNK, NUM_TOKENS * TOKEN_DIM), jnp.float32),
        pltpu.SemaphoreType.DMA,
        pltpu.SemaphoreType.DMA,
        pltpu.SemaphoreType.DMA,
    ],
)(_k1_body)


def _k2_body(bias2_hbm, idx_hbm, out1_hbm, out_hbm,
             idx_v, idx2_v, par_v, b_v0, b_v1, o_v0, o_v1, w_v0, w_v1,
             sem_b0, sem_b1, sem_o0, sem_o1):
    wid = lax.axis_index("s") * NC + lax.axis_index("c")
    base = wid * B_PER_W
    pltpu.sync_copy(idx_hbm.at[pl.ds(base, B_PER_W)], idx_v)

    for i in range(B_PER_W // LANES):
        v = idx_v[pl.ds(i * LANES, LANES)]
        idx2_v[pl.ds(i * LANES, LANES)] = lax.shift_right_logical(v, 1)
        par_v[pl.ds(i * LANES, LANES)] = lax.convert_element_type(
            v & 1, jnp.float32)

    b_bufs = (b_v0, b_v1)
    o_bufs = (o_v0, o_v1)
    w_bufs = (w_v0, w_v1)
    sems_b = (sem_b0, sem_b1)
    sems_o = (sem_o0, sem_o1)

    def start(g, bpar):
        pltpu.async_copy(bias2_hbm.at[idx2_v.at[pl.ds(g * CH2, CH2)]],
                         b_bufs[bpar], sems_b[bpar])
        pltpu.async_copy(
            out1_hbm.at[pl.ds(pl.multiple_of((base + g * CH2) // 2, 8),
                              CH2 // 2)],
            o_bufs[bpar], sems_o[bpar])

    def wait(g, bpar):
        pltpu.make_async_copy(bias2_hbm.at[idx2_v.at[pl.ds(g * CH2, CH2)]],
                              b_bufs[bpar], sems_b[bpar]).wait()
        pltpu.make_async_copy(
            out1_hbm.at[pl.ds(pl.multiple_of((base + g * CH2) // 2, 8),
                              CH2 // 2)],
            o_bufs[bpar], sems_o[bpar]).wait()

    start(0, 0)
    start(1, 1)
    for g in range(N_CHUNKS2):
        bpar = g % 2
        wait(g, bpar)
        b_v = b_bufs[bpar]
        o_v = o_bufs[bpar]
        w_v = w_bufs[bpar]
        goff = g * CH2

        def row_body(r, c2, b_v=b_v, o_v=o_v, w_v=w_v, goff=goff):
            row = goff + r
            pv = par_v[pl.ds(row & ~(LANES - 1), LANES)]
            p = _bcast_lane(pv, row & (LANES - 1))
            half = (r & 1) * TOKEN_DIM
            for c in range(TOKEN_DIM // LANES):
                lo = b_v[r, pl.ds(c * LANES, LANES)]
                hi = b_v[r, pl.ds(TOKEN_DIM + c * LANES, LANES)]
                sv = o_v[lax.shift_right_logical(r, 1),
                         pl.ds(half + c * LANES, LANES)]
                w_v[r, pl.ds(c * LANES, LANES)] = sv + lo + p * (hi - lo)
            return c2

        lax.fori_loop(0, CH2, row_body, 0)
        pltpu.sync_copy(w_v, out_hbm.at[pl.ds(base + goff, CH2)])
        if g + 2 < N_CHUNKS2:
            start(g + 2, bpar)


_k2 = functools.partial(
    pl.kernel,
    out_type=jax.ShapeDtypeStruct((B, TOKEN_DIM), jnp.float32),
    mesh=plsc.VectorSubcoreMesh(core_axis_name="c", subcore_axis_name="s"),
    scratch_types=[
        pltpu.VMEM((B_PER_W,), jnp.int32),
        pltpu.VMEM((B_PER_W,), jnp.int32),
        pltpu.VMEM((B_PER_W,), jnp.float32),
        pltpu.VMEM((CH2, 2 * TOKEN_DIM), jnp.float32),
        pltpu.VMEM((CH2, 2 * TOKEN_DIM), jnp.float32),
        pltpu.VMEM((CH2 // 2, 2 * TOKEN_DIM), jnp.float32),
        pltpu.VMEM((CH2 // 2, 2 * TOKEN_DIM), jnp.float32),
        pltpu.VMEM((CH2, TOKEN_DIM), jnp.float32),
        pltpu.VMEM((CH2, TOKEN_DIM), jnp.float32),
        pltpu.SemaphoreType.DMA,
        pltpu.SemaphoreType.DMA,
        pltpu.SemaphoreType.DMA,
        pltpu.SemaphoreType.DMA,
    ],
)(_k2_body)


def kernel(x, m, style_tokens, style_tokens_bias, W_att, b_att):
    m_i32 = m.astype(jnp.int32)
    alpha_p = _compute_alpha_packed(x, W_att, b_att)
    bias2 = style_tokens_bias.reshape(NUM_VALUES // 2, 2 * TOKEN_DIM)
    out1 = _k1(style_tokens, m_i32, alpha_p)
    return _k2(bias2, m_i32, out1)


# speedup vs baseline: 1.0124x; 1.0124x over previous
import functools

import jax
import jax.numpy as jnp
from jax import lax
from jax.experimental import pallas as pl
from jax.experimental.pallas import tpu as pltpu, tpu_sc as plsc

B = 16384
D_IN = 256
NUM_VALUES = 100000
NUM_TOKENS = 16
TOKEN_DIM = 64
LANES = 16

NC = 2
NS = 16
NW = NC * NS
B_PER_W = B // NW
CHUNK = 32
N_CHUNKS = B_PER_W // CHUNK
CH2 = 128
N_CHUNKS2 = B_PER_W // CH2

_GATHER_DNUMS = lax.GatherDimensionNumbers(
    offset_dims=(), collapsed_slice_dims=(0,), start_index_map=(0,))


def _bcast_lane(vec, t):
    idx = jnp.full((LANES, 1), t, jnp.int32)
    return lax.gather(vec, idx, _GATHER_DNUMS, (1,),
                      mode=lax.GatherScatterMode.PROMISE_IN_BOUNDS)


def _alpha_body(x_ref, w_ref, b_ref, o_ref):
    att = jnp.dot(x_ref[...], w_ref[...], preferred_element_type=jnp.float32)
    o_ref[...] = jax.nn.sigmoid(att + b_ref[...])


def _compute_alpha_packed(x, W_att, b_att):
    blk = 2048
    alpha = pl.pallas_call(
        _alpha_body,
        grid=(B // blk,),
        in_specs=[
            pl.BlockSpec((blk, D_IN), lambda i: (i, 0)),
            pl.BlockSpec((D_IN, NUM_TOKENS), lambda i: (0, 0)),
            pl.BlockSpec((1, NUM_TOKENS), lambda i: (0, 0)),
        ],
        out_specs=pl.BlockSpec((blk, NUM_TOKENS), lambda i: (i, 0)),
        out_shape=jax.ShapeDtypeStruct((B, NUM_TOKENS), jnp.float32),
    )(x, W_att, b_att.reshape(1, NUM_TOKENS))
    return alpha.reshape(B // 8, 8 * NUM_TOKENS)


def _k1_body(tok_hbm, idx_hbm, alpha_hbm, out_hbm,
             idx_v, al_v, out_v, tok_v0, tok_v1, sem_t0, sem_t1):
    wid = lax.axis_index("s") * NC + lax.axis_index("c")
    base = wid * B_PER_W
    pltpu.sync_copy(idx_hbm.at[pl.ds(base, B_PER_W)], idx_v)
    pltpu.sync_copy(
        alpha_hbm.at[pl.ds(pl.multiple_of(base // 8, 8), B_PER_W // 8)], al_v)

    tok_bufs = (tok_v0, tok_v1)
    sems_t = (sem_t0, sem_t1)

    def start(g, bpar):
        pltpu.async_copy(tok_hbm.at[idx_v.at[pl.ds(g * CHUNK, CHUNK)]],
                         tok_bufs[bpar], sems_t[bpar])

    def wait(g, bpar):
        pltpu.make_async_copy(tok_hbm.at[idx_v.at[pl.ds(g * CHUNK, CHUNK)]],
                              tok_bufs[bpar], sems_t[bpar]).wait()

    start(0, 0)
    start(1, 1)

    def outer(o, carry):
        for bpar in (0, 1):
            g = 2 * o + bpar
            wait(g, bpar)
            tok_v = tok_bufs[bpar]
            goff = g * CHUNK

            def row_body(r, c2, goff=goff, tok_v=tok_v):
                row = goff + r
                a_vec = al_v[lax.shift_right_logical(row, 3),
                             pl.ds((row & 7) * LANES, LANES)]
                a = [_bcast_lane(a_vec, t) for t in range(NUM_TOKENS)]
                half = (r & 1) * TOKEN_DIM
                for c in range(TOKEN_DIM // LANES):
                    acc = a[0] * tok_v[r, pl.ds(c * LANES, LANES)]
                    for t in range(1, NUM_TOKENS):
                        acc = acc + a[t] * tok_v[r, pl.ds(t * TOKEN_DIM + c * LANES, LANES)]
                    out_v[lax.shift_right_logical(r, 1),
                          pl.ds(half + c * LANES, LANES)] = acc
                return c2

            lax.fori_loop(0, CHUNK, row_body, 0)
            pltpu.sync_copy(
                out_v,
                out_hbm.at[pl.ds(pl.multiple_of((base + goff) // 2, 8),
                                 CHUNK // 2)])
            start((g + 2) & (N_CHUNKS - 1), bpar)
        return carry

    lax.fori_loop(0, N_CHUNKS // 2, outer, 0)
    wait(0, 0)
    wait(1, 1)


_k1 = functools.partial(
    pl.kernel,
    out_type=jax.ShapeDtypeStruct((B // 2, 2 * TOKEN_DIM), jnp.float32),
    mesh=plsc.VectorSubcoreMesh(core_axis_name="c", subcore_axis_name="s"),
    scratch_types=[
        pltpu.VMEM((B_PER_W,), jnp.int32),
        pltpu.VMEM((B_PER_W // 8, 8 * NUM_TOKENS), jnp.float32),
        pltpu.VMEM((CHUNK // 2, 2 * TOKEN_DIM), jnp.float32),
        pltpu.VMEM((CHUNK, NUM_TOKENS * TOKEN_DIM), jnp.float32),
        pltpu.VMEM((CHUNK, NUM_TOKENS * TOKEN_DIM), jnp.float32),
        pltpu.SemaphoreType.DMA,
        pltpu.SemaphoreType.DMA,
    ],
)(_k1_body)


def _k2_body(bias_hbm, idx_hbm, out1_hbm, out_hbm,
             idx_v, b_v0, b_v1, o_v0, o_v1, w_v0, w_v1,
             sem_b0, sem_b1, sem_o0, sem_o1):
    wid = lax.axis_index("s") * NC + lax.axis_index("c")
    base = wid * B_PER_W
    pltpu.sync_copy(idx_hbm.at[pl.ds(base, B_PER_W)], idx_v)

    b_bufs = (b_v0, b_v1)
    o_bufs = (o_v0, o_v1)
    w_bufs = (w_v0, w_v1)
    sems_b = (sem_b0, sem_b1)
    sems_o = (sem_o0, sem_o1)

    def start(g, bpar):
        pltpu.async_copy(bias_hbm.at[idx_v.at[pl.ds(g * CH2, CH2)]],
                         b_bufs[bpar], sems_b[bpar])
        pltpu.async_copy(
            out1_hbm.at[pl.ds(pl.multiple_of((base + g * CH2) // 2, 8),
                              CH2 // 2)],
            o_bufs[bpar], sems_o[bpar])

    def wait(g, bpar):
        pltpu.make_async_copy(bias_hbm.at[idx_v.at[pl.ds(g * CH2, CH2)]],
                              b_bufs[bpar], sems_b[bpar]).wait()
        pltpu.make_async_copy(
            out1_hbm.at[pl.ds(pl.multiple_of((base + g * CH2) // 2, 8),
                              CH2 // 2)],
            o_bufs[bpar], sems_o[bpar]).wait()

    start(0, 0)
    start(1, 1)
    for g in range(N_CHUNKS2):
        bpar = g % 2
        wait(g, bpar)
        b_v = b_bufs[bpar]
        o_v = o_bufs[bpar]
        w_v = w_bufs[bpar]
        goff = g * CH2

        def row_body(r, c2, b_v=b_v, o_v=o_v, w_v=w_v):
            half = (r & 1) * TOKEN_DIM
            for c in range(TOKEN_DIM // LANES):
                sv = o_v[lax.shift_right_logical(r, 1),
                         pl.ds(half + c * LANES, LANES)]
                w_v[r, pl.ds(c * LANES, LANES)] = (
                    sv + b_v[r, pl.ds(c * LANES, LANES)])
            return c2

        lax.fori_loop(0, CH2, row_body, 0)
        pltpu.sync_copy(w_v, out_hbm.at[pl.ds(base + goff, CH2)])
        if g + 2 < N_CHUNKS2:
            start(g + 2, bpar)


_k2 = functools.partial(
    pl.kernel,
    out_type=jax.ShapeDtypeStruct((B, TOKEN_DIM), jnp.float32),
    mesh=plsc.VectorSubcoreMesh(core_axis_name="c", subcore_axis_name="s"),
    scratch_types=[
        pltpu.VMEM((B_PER_W,), jnp.int32),
        pltpu.VMEM((CH2, 2 * TOKEN_DIM), jnp.float32),
        pltpu.VMEM((CH2, 2 * TOKEN_DIM), jnp.float32),
        pltpu.VMEM((CH2 // 2, 2 * TOKEN_DIM), jnp.float32),
        pltpu.VMEM((CH2 // 2, 2 * TOKEN_DIM), jnp.float32),
        pltpu.VMEM((CH2, TOKEN_DIM), jnp.float32),
        pltpu.VMEM((CH2, TOKEN_DIM), jnp.float32),
        pltpu.SemaphoreType.DMA,
        pltpu.SemaphoreType.DMA,
        pltpu.SemaphoreType.DMA,
        pltpu.SemaphoreType.DMA,
    ],
)(_k2_body)


def kernel(x, m, style_tokens, style_tokens_bias, W_att, b_att):
    m_i32 = m.astype(jnp.int32)
    alpha_p = _compute_alpha_packed(x, W_att, b_att)
    bias128 = jnp.pad(style_tokens_bias, ((0, 0), (0, TOKEN_DIM)))
    out1 = _k1(style_tokens, m_i32, alpha_p)
    return _k2(bias128, m_i32, out1)


# speedup vs baseline: 1.0745x; 1.0613x over previous
import functools

import jax
import jax.numpy as jnp
from jax import lax
from jax.experimental import pallas as pl
from jax.experimental.pallas import tpu as pltpu, tpu_sc as plsc

B = 16384
D_IN = 256
NUM_VALUES = 100000
NUM_TOKENS = 16
TOKEN_DIM = 64
LANES = 16

NC = 2
NS = 16
NW = NC * NS
B_PER_W = B // NW
CHUNK = 32
N_CHUNKS = B_PER_W // CHUNK
CH2 = 128
N_CHUNKS2 = B_PER_W // CH2

_GATHER_DNUMS = lax.GatherDimensionNumbers(
    offset_dims=(), collapsed_slice_dims=(0,), start_index_map=(0,))


def _bcast_lane(vec, t):
    idx = jnp.full((LANES, 1), t, jnp.int32)
    return lax.gather(vec, idx, _GATHER_DNUMS, (1,),
                      mode=lax.GatherScatterMode.PROMISE_IN_BOUNDS)


def _alpha_body(x_ref, w_ref, b_ref, o_ref):
    att = jnp.dot(x_ref[...], w_ref[...], preferred_element_type=jnp.float32)
    o_ref[...] = jax.nn.sigmoid(att + b_ref[...])


def _compute_alpha_packed(x, W_att, b_att):
    blk = 2048
    alpha = pl.pallas_call(
        _alpha_body,
        grid=(B // blk,),
        in_specs=[
            pl.BlockSpec((blk, D_IN), lambda i: (i, 0)),
            pl.BlockSpec((D_IN, NUM_TOKENS), lambda i: (0, 0)),
            pl.BlockSpec((1, NUM_TOKENS), lambda i: (0, 0)),
        ],
        out_specs=pl.BlockSpec((blk, NUM_TOKENS), lambda i: (i, 0)),
        out_shape=jax.ShapeDtypeStruct((B, NUM_TOKENS), jnp.float32),
    )(x, W_att, b_att.reshape(1, NUM_TOKENS))
    return alpha.reshape(B // 8, 8 * NUM_TOKENS)


def _k1_body(tok_hbm, idx_hbm, alpha_hbm, out_hbm,
             idx_v, al_v, out_v, tok_v0, tok_v1, sem_t0, sem_t1):
    wid = lax.axis_index("s") * NC + lax.axis_index("c")
    base = wid * B_PER_W
    pltpu.sync_copy(idx_hbm.at[pl.ds(base, B_PER_W)], idx_v)
    pltpu.sync_copy(
        alpha_hbm.at[pl.ds(pl.multiple_of(base // 8, 8), B_PER_W // 8)], al_v)

    tok_bufs = (tok_v0, tok_v1)
    sems_t = (sem_t0, sem_t1)

    def start(g, bpar):
        pltpu.async_copy(tok_hbm.at[idx_v.at[pl.ds(g * CHUNK, CHUNK)]],
                         tok_bufs[bpar], sems_t[bpar])

    def wait(g, bpar):
        pltpu.make_async_copy(tok_hbm.at[idx_v.at[pl.ds(g * CHUNK, CHUNK)]],
                              tok_bufs[bpar], sems_t[bpar]).wait()

    start(0, 0)
    start(1, 1)

    def outer(o, carry):
        for bpar in (0, 1):
            g = 2 * o + bpar
            wait(g, bpar)
            tok_v = tok_bufs[bpar]
            goff = g * CHUNK

            def row_body(r, c2, goff=goff, tok_v=tok_v):
                row = goff + r
                a_vec = al_v[lax.shift_right_logical(row, 3),
                             pl.ds((row & 7) * LANES, LANES)]
                a = [_bcast_lane(a_vec, t) for t in range(NUM_TOKENS)]
                half = (r & 1) * TOKEN_DIM
                for c in range(TOKEN_DIM // LANES):
                    acc = a[0] * tok_v[r, pl.ds(c * LANES, LANES)]
                    for t in range(1, NUM_TOKENS):
                        acc = acc + a[t] * tok_v[r, pl.ds(t * TOKEN_DIM + c * LANES, LANES)]
                    out_v[lax.shift_right_logical(r, 1),
                          pl.ds(half + c * LANES, LANES)] = acc
                return c2

            lax.fori_loop(0, CHUNK, row_body, 0)
            pltpu.sync_copy(
                out_v,
                out_hbm.at[pl.ds(pl.multiple_of((base + goff) // 2, 8),
                                 CHUNK // 2)])
            start((g + 2) & (N_CHUNKS - 1), bpar)
        return carry

    lax.fori_loop(0, N_CHUNKS // 2, outer, 0)
    wait(0, 0)
    wait(1, 1)


_k1 = functools.partial(
    pl.kernel,
    out_type=jax.ShapeDtypeStruct((B // 2, 2 * TOKEN_DIM), jnp.float32),
    mesh=plsc.VectorSubcoreMesh(core_axis_name="c", subcore_axis_name="s"),
    scratch_types=[
        pltpu.VMEM((B_PER_W,), jnp.int32),
        pltpu.VMEM((B_PER_W // 8, 8 * NUM_TOKENS), jnp.float32),
        pltpu.VMEM((CHUNK // 2, 2 * TOKEN_DIM), jnp.float32),
        pltpu.VMEM((CHUNK, NUM_TOKENS * TOKEN_DIM), jnp.float32),
        pltpu.VMEM((CHUNK, NUM_TOKENS * TOKEN_DIM), jnp.float32),
        pltpu.SemaphoreType.DMA,
        pltpu.SemaphoreType.DMA,
    ],
)(_k1_body)


def _k2_body(bias2_hbm, idx_hbm, out1_hbm, out_hbm,
             idx_v, idx2_v, par_v, b_v0, b_v1, o_v0, o_v1, w_v0, w_v1,
             sem_b0, sem_b1, sem_o0, sem_o1):
    wid = lax.axis_index("s") * NC + lax.axis_index("c")
    base = wid * B_PER_W
    pltpu.sync_copy(idx_hbm.at[pl.ds(base, B_PER_W)], idx_v)

    for i in range(B_PER_W // LANES):
        v = idx_v[pl.ds(i * LANES, LANES)]
        idx2_v[pl.ds(i * LANES, LANES)] = lax.shift_right_logical(v, 1)
        par_v[pl.ds(i * LANES, LANES)] = lax.convert_element_type(
            v & 1, jnp.float32)

    b_bufs = (b_v0, b_v1)
    o_bufs = (o_v0, o_v1)
    w_bufs = (w_v0, w_v1)
    sems_b = (sem_b0, sem_b1)
    sems_o = (sem_o0, sem_o1)

    def start(g, bpar):
        pltpu.async_copy(bias2_hbm.at[idx2_v.at[pl.ds(g * CH2, CH2)]],
                         b_bufs[bpar], sems_b[bpar])
        pltpu.async_copy(
            out1_hbm.at[pl.ds(pl.multiple_of((base + g * CH2) // 2, 8),
                              CH2 // 2)],
            o_bufs[bpar], sems_o[bpar])

    def wait(g, bpar):
        pltpu.make_async_copy(bias2_hbm.at[idx2_v.at[pl.ds(g * CH2, CH2)]],
                              b_bufs[bpar], sems_b[bpar]).wait()
        pltpu.make_async_copy(
            out1_hbm.at[pl.ds(pl.multiple_of((base + g * CH2) // 2, 8),
                              CH2 // 2)],
            o_bufs[bpar], sems_o[bpar]).wait()

    start(0, 0)
    start(1, 1)
    for g in range(N_CHUNKS2):
        bpar = g % 2
        wait(g, bpar)
        b_v = b_bufs[bpar]
        o_v = o_bufs[bpar]
        w_v = w_bufs[bpar]
        goff = g * CH2

        def row_body(r, c2, b_v=b_v, o_v=o_v, w_v=w_v, goff=goff):
            row = goff + r
            pv = par_v[pl.ds(row & ~(LANES - 1), LANES)]
            p = _bcast_lane(pv, row & (LANES - 1))
            half = (r & 1) * TOKEN_DIM
            for c in range(TOKEN_DIM // LANES):
                lo = b_v[r, pl.ds(c * LANES, LANES)]
                hi = b_v[r, pl.ds(TOKEN_DIM + c * LANES, LANES)]
                sv = o_v[lax.shift_right_logical(r, 1),
                         pl.ds(half + c * LANES, LANES)]
                w_v[r, pl.ds(c * LANES, LANES)] = sv + lo + p * (hi - lo)
            return c2

        lax.fori_loop(0, CH2, row_body, 0)
        pltpu.sync_copy(w_v, out_hbm.at[pl.ds(base + goff, CH2)])
        if g + 2 < N_CHUNKS2:
            start(g + 2, bpar)


_k2 = functools.partial(
    pl.kernel,
    out_type=jax.ShapeDtypeStruct((B, TOKEN_DIM), jnp.float32),
    mesh=plsc.VectorSubcoreMesh(core_axis_name="c", subcore_axis_name="s"),
    scratch_types=[
        pltpu.VMEM((B_PER_W,), jnp.int32),
        pltpu.VMEM((B_PER_W,), jnp.int32),
        pltpu.VMEM((B_PER_W,), jnp.float32),
        pltpu.VMEM((CH2, 2 * TOKEN_DIM), jnp.float32),
        pltpu.VMEM((CH2, 2 * TOKEN_DIM), jnp.float32),
        pltpu.VMEM((CH2 // 2, 2 * TOKEN_DIM), jnp.float32),
        pltpu.VMEM((CH2 // 2, 2 * TOKEN_DIM), jnp.float32),
        pltpu.VMEM((CH2, TOKEN_DIM), jnp.float32),
        pltpu.VMEM((CH2, TOKEN_DIM), jnp.float32),
        pltpu.SemaphoreType.DMA,
        pltpu.SemaphoreType.DMA,
        pltpu.SemaphoreType.DMA,
        pltpu.SemaphoreType.DMA,
    ],
)(_k2_body)


def kernel(x, m, style_tokens, style_tokens_bias, W_att, b_att):
    m_i32 = m.astype(jnp.int32)
    alpha_p = _compute_alpha_packed(x, W_att, b_att)
    bias2 = style_tokens_bias.reshape(NUM_VALUES // 2, 2 * TOKEN_DIM)
    out1 = _k1(style_tokens, m_i32, alpha_p)
    return _k2(bias2, m_i32, out1)


# speedup vs baseline: 1.0861x; 1.0109x over previous
import functools

import jax
import jax.numpy as jnp
from jax import lax
from jax.experimental import pallas as pl
from jax.experimental.pallas import tpu as pltpu, tpu_sc as plsc

B = 16384
D_IN = 256
NUM_VALUES = 100000
NUM_TOKENS = 16
TOKEN_DIM = 64
LANES = 16

NC = 2
NS = 16
NW = NC * NS
B_PER_W = B // NW
CHUNK = 32
N_CHUNKS = B_PER_W // CHUNK
CH2 = 64
N_CHUNKS2 = B_PER_W // CH2

_GATHER_DNUMS = lax.GatherDimensionNumbers(
    offset_dims=(), collapsed_slice_dims=(0,), start_index_map=(0,))


def _bcast_lane(vec, t):
    idx = jnp.full((LANES, 1), t, jnp.int32)
    return lax.gather(vec, idx, _GATHER_DNUMS, (1,),
                      mode=lax.GatherScatterMode.PROMISE_IN_BOUNDS)


def _alpha_body(x_ref, w_ref, b_ref, o_ref):
    att = jnp.dot(x_ref[...], w_ref[...], preferred_element_type=jnp.float32)
    o_ref[...] = jax.nn.sigmoid(att + b_ref[...])


def _compute_alpha_packed(x, W_att, b_att):
    blk = 2048
    alpha = pl.pallas_call(
        _alpha_body,
        grid=(B // blk,),
        in_specs=[
            pl.BlockSpec((blk, D_IN), lambda i: (i, 0)),
            pl.BlockSpec((D_IN, NUM_TOKENS), lambda i: (0, 0)),
            pl.BlockSpec((1, NUM_TOKENS), lambda i: (0, 0)),
        ],
        out_specs=pl.BlockSpec((blk, NUM_TOKENS), lambda i: (i, 0)),
        out_shape=jax.ShapeDtypeStruct((B, NUM_TOKENS), jnp.float32),
    )(x, W_att, b_att.reshape(1, NUM_TOKENS))
    return alpha.reshape(B // 8, 8 * NUM_TOKENS)


def _k1_body(tok_hbm, idx_hbm, alpha_hbm, out_hbm,
             idx_v, al_v, out_v, tok_v0, tok_v1, sem_t0, sem_t1):
    wid = lax.axis_index("s") * NC + lax.axis_index("c")
    base = wid * B_PER_W
    pltpu.sync_copy(idx_hbm.at[pl.ds(base, B_PER_W)], idx_v)
    pltpu.sync_copy(
        alpha_hbm.at[pl.ds(pl.multiple_of(base // 8, 8), B_PER_W // 8)], al_v)

    tok_bufs = (tok_v0, tok_v1)
    sems_t = (sem_t0, sem_t1)

    def start(g, bpar):
        pltpu.async_copy(tok_hbm.at[idx_v.at[pl.ds(g * CHUNK, CHUNK)]],
                         tok_bufs[bpar], sems_t[bpar])

    def wait(g, bpar):
        pltpu.make_async_copy(tok_hbm.at[idx_v.at[pl.ds(g * CHUNK, CHUNK)]],
                              tok_bufs[bpar], sems_t[bpar]).wait()

    start(0, 0)
    start(1, 1)

    def outer(o, carry):
        for bpar in (0, 1):
            g = 2 * o + bpar
            wait(g, bpar)
            tok_v = tok_bufs[bpar]
            goff = g * CHUNK

            def row_body(r, c2, goff=goff, tok_v=tok_v):
                row = goff + r
                a_vec = al_v[lax.shift_right_logical(row, 3),
                             pl.ds((row & 7) * LANES, LANES)]
                a = [_bcast_lane(a_vec, t) for t in range(NUM_TOKENS)]
                half = (r & 1) * TOKEN_DIM
                for c in range(TOKEN_DIM // LANES):
                    acc = a[0] * tok_v[r, pl.ds(c * LANES, LANES)]
                    for t in range(1, NUM_TOKENS):
                        acc = acc + a[t] * tok_v[r, pl.ds(t * TOKEN_DIM + c * LANES, LANES)]
                    out_v[lax.shift_right_logical(r, 1),
                          pl.ds(half + c * LANES, LANES)] = acc
                return c2

            lax.fori_loop(0, CHUNK, row_body, 0)
            pltpu.sync_copy(
                out_v,
                out_hbm.at[pl.ds(pl.multiple_of((base + goff) // 2, 8),
                                 CHUNK // 2)])

            @pl.when(g + 2 < N_CHUNKS)
            def _():
                start(g + 2, bpar)
        return carry

    lax.fori_loop(0, N_CHUNKS // 2, outer, 0)


_k1 = functools.partial(
    pl.kernel,
    out_type=jax.ShapeDtypeStruct((B // 2, 2 * TOKEN_DIM), jnp.float32),
    mesh=plsc.VectorSubcoreMesh(core_axis_name="c", subcore_axis_name="s"),
    scratch_types=[
        pltpu.VMEM((B_PER_W,), jnp.int32),
        pltpu.VMEM((B_PER_W // 8, 8 * NUM_TOKENS), jnp.float32),
        pltpu.VMEM((CHUNK // 2, 2 * TOKEN_DIM), jnp.float32),
        pltpu.VMEM((CHUNK, NUM_TOKENS * TOKEN_DIM), jnp.float32),
        pltpu.VMEM((CHUNK, NUM_TOKENS * TOKEN_DIM), jnp.float32),
        pltpu.SemaphoreType.DMA,
        pltpu.SemaphoreType.DMA,
    ],
)(_k1_body)


def _k2_body(bias2_hbm, idx_hbm, out1_hbm, out_hbm,
             idx_v, idx2_v, par_v, b_v0, b_v1, o_v0, o_v1, w_v0, w_v1,
             sem_b0, sem_b1, sem_o0, sem_o1):
    wid = lax.axis_index("s") * NC + lax.axis_index("c")
    base = wid * B_PER_W
    pltpu.sync_copy(idx_hbm.at[pl.ds(base, B_PER_W)], idx_v)

    for i in range(B_PER_W // LANES):
        v = idx_v[pl.ds(i * LANES, LANES)]
        idx2_v[pl.ds(i * LANES, LANES)] = lax.shift_right_logical(v, 1)
        par_v[pl.ds(i * LANES, LANES)] = lax.convert_element_type(
            v & 1, jnp.float32)

    b_bufs = (b_v0, b_v1)
    o_bufs = (o_v0, o_v1)
    w_bufs = (w_v0, w_v1)
    sems_b = (sem_b0, sem_b1)
    sems_o = (sem_o0, sem_o1)

    def start(g, bpar):
        pltpu.async_copy(bias2_hbm.at[idx2_v.at[pl.ds(g * CH2, CH2)]],
                         b_bufs[bpar], sems_b[bpar])
        pltpu.async_copy(
            out1_hbm.at[pl.ds(pl.multiple_of((base + g * CH2) // 2, 8),
                              CH2 // 2)],
            o_bufs[bpar], sems_o[bpar])

    def wait(g, bpar):
        pltpu.make_async_copy(bias2_hbm.at[idx2_v.at[pl.ds(g * CH2, CH2)]],
                              b_bufs[bpar], sems_b[bpar]).wait()
        pltpu.make_async_copy(
            out1_hbm.at[pl.ds(pl.multiple_of((base + g * CH2) // 2, 8),
                              CH2 // 2)],
            o_bufs[bpar], sems_o[bpar]).wait()

    start(0, 0)
    start(1, 1)
    for g in range(N_CHUNKS2):
        bpar = g % 2
        wait(g, bpar)
        b_v = b_bufs[bpar]
        o_v = o_bufs[bpar]
        w_v = w_bufs[bpar]
        goff = g * CH2

        def row_body(r, c2, b_v=b_v, o_v=o_v, w_v=w_v, goff=goff):
            row = goff + r
            pv = par_v[pl.ds(row & ~(LANES - 1), LANES)]
            p = _bcast_lane(pv, row & (LANES - 1))
            half = (r & 1) * TOKEN_DIM
            for c in range(TOKEN_DIM // LANES):
                lo = b_v[r, pl.ds(c * LANES, LANES)]
                hi = b_v[r, pl.ds(TOKEN_DIM + c * LANES, LANES)]
                sv = o_v[lax.shift_right_logical(r, 1),
                         pl.ds(half + c * LANES, LANES)]
                w_v[r, pl.ds(c * LANES, LANES)] = sv + lo + p * (hi - lo)
            return c2

        lax.fori_loop(0, CH2, row_body, 0)
        pltpu.sync_copy(w_v, out_hbm.at[pl.ds(base + goff, CH2)])
        if g + 2 < N_CHUNKS2:
            start(g + 2, bpar)


_k2 = functools.partial(
    pl.kernel,
    out_type=jax.ShapeDtypeStruct((B, TOKEN_DIM), jnp.float32),
    mesh=plsc.VectorSubcoreMesh(core_axis_name="c", subcore_axis_name="s"),
    scratch_types=[
        pltpu.VMEM((B_PER_W,), jnp.int32),
        pltpu.VMEM((B_PER_W,), jnp.int32),
        pltpu.VMEM((B_PER_W,), jnp.float32),
        pltpu.VMEM((CH2, 2 * TOKEN_DIM), jnp.float32),
        pltpu.VMEM((CH2, 2 * TOKEN_DIM), jnp.float32),
        pltpu.VMEM((CH2 // 2, 2 * TOKEN_DIM), jnp.float32),
        pltpu.VMEM((CH2 // 2, 2 * TOKEN_DIM), jnp.float32),
        pltpu.VMEM((CH2, TOKEN_DIM), jnp.float32),
        pltpu.VMEM((CH2, TOKEN_DIM), jnp.float32),
        pltpu.SemaphoreType.DMA,
        pltpu.SemaphoreType.DMA,
        pltpu.SemaphoreType.DMA,
        pltpu.SemaphoreType.DMA,
    ],
)(_k2_body)


def kernel(x, m, style_tokens, style_tokens_bias, W_att, b_att):
    m_i32 = m.astype(jnp.int32)
    alpha_p = _compute_alpha_packed(x, W_att, b_att)
    bias2 = style_tokens_bias.reshape(NUM_VALUES // 2, 2 * TOKEN_DIM)
    out1 = _k1(style_tokens, m_i32, alpha_p)
    return _k2(bias2, m_i32, out1)
